# baseline (speedup 1.0000x reference)
import functools

import jax
import jax.numpy as jnp
from jax import lax
from jax.experimental import pallas as pl
from jax.experimental.pallas import tpu as pltpu
from jax.experimental.pallas import tpu_sc as plsc

_NUM_CORES = 2
_NUM_SUBCORES = 16
_NUM_WORKERS = _NUM_CORES * _NUM_SUBCORES
_L = 16


@functools.lru_cache(maxsize=None)
def _build(B, V, D, n_chunks):
    b_per_w = B // _NUM_WORKERS
    chunk = b_per_w // n_chunks
    mesh = plsc.VectorSubcoreMesh(core_axis_name="c", subcore_axis_name="s")

    @functools.partial(
        pl.kernel,
        mesh=mesh,
        out_type=jax.ShapeDtypeStruct((B, D), jnp.float32),
        scratch_types=[
            pltpu.VMEM((b_per_w,), jnp.int32),
            pltpu.VMEM((b_per_w,), jnp.int32),
            pltpu.VMEM((2, chunk, 2 * D), jnp.float32),
            pltpu.VMEM((2, chunk, D), jnp.float32),
            pltpu.SemaphoreType.DMA,
        ]
        + [pltpu.SemaphoreType.DMA] * (2 * n_chunks),
    )
    def gather_kernel(idx_hbm, table_hbm, out_hbm, idx_v, pair_v, rows_v,
                      cmp_v, isem, *sems):
        gsems, osems = sems[:n_chunks], sems[n_chunks:]
        wid = lax.axis_index("s") * _NUM_CORES + lax.axis_index("c")
        base = wid * b_per_w
        pltpu.async_copy(idx_hbm.at[pl.ds(base, b_per_w)], idx_v, isem).wait()

        for v in range(b_per_w // _L):
            sl = pl.ds(v * _L, _L)
            pair_v[sl] = jax.lax.shift_right_logical(idx_v[sl], 1)

        def gather_start(c, slot):
            return pltpu.async_copy(
                table_hbm.at[pair_v.at[pl.ds(c * chunk, chunk)]],
                rows_v.at[slot],
                gsems[c],
            )

        def compact(c, slot):
            def body(g, _):
                tokv = idx_v[pl.ds(c * chunk + g * _L, _L)]
                offv = (tokv & 1) * D
                for t2 in range(_L):
                    off = offv[t2]
                    t = g * _L + t2
                    for k in range(D // _L):
                        cmp_v[slot, t, pl.ds(k * _L, _L)] = rows_v[
                            slot, t, pl.ds(off + k * _L, _L)
                        ]
                return 0

            lax.fori_loop(0, chunk // _L, body, 0)

        g0 = gather_start(0, 0)
        outs = []
        for c in range(n_chunks):
            slot = c % 2
            if c + 1 < n_chunks:
                gn = gather_start(c + 1, 1 - slot)
            g0.wait()
            if c >= 2:
                outs[c - 2].wait()
            compact(c, slot)
            outs.append(
                pltpu.async_copy(
                    cmp_v.at[slot],
                    out_hbm.at[pl.ds(base + c * chunk, chunk)],
                    osems[c],
                )
            )
            if c + 1 < n_chunks:
                g0 = gn
        for o in outs[-2:]:
            o.wait()

    return gather_kernel


def kernel(token_ids, E):
    B = token_ids.shape[0]
    V, D = E.shape
    idx = token_ids.astype(jnp.int32)
    E2 = E.reshape(V // 2, 2 * D)
    return _build(B, V, D, 4)(idx, E2)


# speedup vs baseline: 1.5731x; 1.5731x over previous
import functools

import jax
import jax.numpy as jnp
from jax import lax
from jax.experimental import pallas as pl
from jax.experimental.pallas import tpu as pltpu
from jax.experimental.pallas import tpu_sc as plsc

_NUM_CORES = 2
_NUM_SUBCORES = 16
_NUM_WORKERS = _NUM_CORES * _NUM_SUBCORES
_L = 16


@functools.lru_cache(maxsize=None)
def _build(B, V, D):
    b_per_w = B // _NUM_WORKERS
    n_pairs = b_per_w // (2 * _L)
    mesh = plsc.VectorSubcoreMesh(core_axis_name="c", subcore_axis_name="s")

    @functools.partial(
        pl.kernel,
        mesh=mesh,
        out_type=jax.ShapeDtypeStruct((B, D), jnp.float32),
        scratch_types=[
            pltpu.VMEM((b_per_w,), jnp.int32),
            pltpu.VMEM((2, _L, 8, D), jnp.float32),
            pltpu.VMEM((b_per_w, D), jnp.float32),
            pltpu.SemaphoreType.DMA,
            pltpu.SemaphoreType.DMA,
            pltpu.SemaphoreType.DMA,
            pltpu.SemaphoreType.DMA,
        ],
    )
    def gather_kernel(idx_hbm, table_hbm, out_hbm, idx_v, land_v, row_v,
                      isem, osem, gsem0, gsem1):
        wid = lax.axis_index("s") * _NUM_CORES + lax.axis_index("c")
        base = wid * b_per_w
        pltpu.async_copy(idx_hbm.at[pl.ds(base, b_per_w)], idx_v, isem).wait()

        def fire(g, buf, sem):
            tokv = idx_v[pl.ds(g * _L, _L)]
            basev = tokv - (tokv & 7)
            for j in range(_L):
                start = pl.multiple_of(basev[j], 8)
                pltpu.async_copy(
                    table_hbm.at[pl.ds(start, 8), :],
                    land_v.at[buf, j],
                    sem,
                )

        def drain(sem):
            for _ in range(_L):
                pltpu.make_async_copy(
                    table_hbm.at[pl.ds(0, 8), :],
                    land_v.at[0, 0],
                    sem,
                ).wait()

        def select(g, buf):
            tokv = idx_v[pl.ds(g * _L, _L)]
            rv = tokv & 7
            for j in range(_L):
                r = rv[j]
                t = g * _L + j
                for k in range(D // _L):
                    row_v[t, pl.ds(k * _L, _L)] = land_v[
                        buf, j, r, pl.ds(k * _L, _L)
                    ]

        def body(i, _):
            fire(2 * i, 0, gsem0)
            fire(2 * i + 1, 1, gsem1)
            drain(gsem0)
            select(2 * i, 0)
            drain(gsem1)
            select(2 * i + 1, 1)
            return 0

        lax.fori_loop(0, n_pairs, body, 0)

        pltpu.async_copy(row_v, out_hbm.at[pl.ds(base, b_per_w)], osem).wait()

    return gather_kernel


def kernel(token_ids, E):
    B = token_ids.shape[0]
    V, D = E.shape
    idx = token_ids.astype(jnp.int32)
    return _build(B, V, D)(idx, E)


# speedup vs baseline: 2.2526x; 1.4319x over previous
import functools

import jax
import jax.numpy as jnp
from jax import lax
from jax.experimental import pallas as pl
from jax.experimental.pallas import tpu as pltpu
from jax.experimental.pallas import tpu_sc as plsc

_NUM_CORES = 2
_NUM_SUBCORES = 16
_NUM_WORKERS = _NUM_CORES * _NUM_SUBCORES
_L = 16
_CHUNK = 16


@functools.lru_cache(maxsize=None)
def _build(B, V, D):
    b_per_w = B // _NUM_WORKERS
    n_iters = b_per_w // (2 * _CHUNK)
    mesh = plsc.VectorSubcoreMesh(core_axis_name="c", subcore_axis_name="s")

    @functools.partial(
        pl.kernel,
        mesh=mesh,
        out_type=jax.ShapeDtypeStruct((B, D), jnp.float32),
        scratch_types=[
            pltpu.VMEM((b_per_w,), jnp.int32),
            pltpu.VMEM((2, _CHUNK, 8, D), jnp.float32),
            pltpu.VMEM((2 * _CHUNK, D), jnp.float32),
            pltpu.SemaphoreType.DMA,
            pltpu.SemaphoreType.DMA,
            pltpu.SemaphoreType.DMA,
            pltpu.SemaphoreType.DMA,
        ],
    )
    def gather_kernel(idx_hbm, table_hbm, out_hbm, idx_v, land_v, row_v,
                      isem, gsem0, gsem1, osem):
        gsems = (gsem0, gsem1)
        wid = lax.axis_index("s") * _NUM_CORES + lax.axis_index("c")
        base = wid * b_per_w
        pltpu.async_copy(idx_hbm.at[pl.ds(base, b_per_w)], idx_v, isem).wait()

        def fire(c, buf):
            for g in range(_CHUNK // _L):
                grpv = jax.lax.shift_right_logical(
                    idx_v[pl.ds(c * _CHUNK + g * _L, _L)], 3
                )
                for j in range(_L):
                    pltpu.async_copy(
                        table_hbm.at[grpv[j]],
                        land_v.at[buf, g * _L + j],
                        gsems[buf],
                    )

        def drain(buf):
            for _ in range(_CHUNK):
                pltpu.make_async_copy(
                    table_hbm.at[0], land_v.at[0, 0], gsems[buf]
                ).wait()

        def owait():
            pltpu.make_async_copy(
                row_v, out_hbm.at[pl.ds(0, 2 * _CHUNK)], osem
            ).wait()

        def select(c, buf, half):
            for g in range(_CHUNK // _L):
                tokv = idx_v[pl.ds(c * _CHUNK + g * _L, _L)]
                rv = tokv & 7
                for j in range(_L):
                    r = rv[j]
                    t = half * _CHUNK + g * _L + j
                    for k in range(D // _L):
                        row_v[t, pl.ds(k * _L, _L)] = land_v[
                            buf, g * _L + j, r, pl.ds(k * _L, _L)
                        ]

        def body(i, _):
            fire(2 * i, 0)
            fire(2 * i + 1, 1)

            @pl.when(i >= 1)
            def _():
                owait()

            drain(0)
            select(2 * i, 0, 0)
            drain(1)
            select(2 * i + 1, 1, 1)
            dst = pl.multiple_of(base + i * 2 * _CHUNK, 8)
            pltpu.async_copy(
                row_v, out_hbm.at[pl.ds(dst, 2 * _CHUNK)], osem
            )
            return 0

        lax.fori_loop(0, n_iters, body, 0)
        owait()

    return gather_kernel


def kernel(token_ids, E):
    B = token_ids.shape[0]
    V, D = E.shape
    idx = token_ids.astype(jnp.int32)
    E3 = E.reshape(V // 8, 8, D)
    return _build(B, V, D)(idx, E3)


# speedup vs baseline: 2.2995x; 1.0209x over previous
import functools

import jax
import jax.numpy as jnp
from jax import lax
from jax.experimental import pallas as pl
from jax.experimental.pallas import tpu as pltpu
from jax.experimental.pallas import tpu_sc as plsc

_NUM_CORES = 2
_NUM_SUBCORES = 16
_NUM_WORKERS = _NUM_CORES * _NUM_SUBCORES
_L = 16
_CHUNK = 16


@functools.lru_cache(maxsize=None)
def _build(B, V, D):
    b_per_w = B // _NUM_WORKERS
    n_iters = b_per_w // (2 * _CHUNK)
    mesh = plsc.VectorSubcoreMesh(core_axis_name="c", subcore_axis_name="s")

    @functools.partial(
        pl.kernel,
        mesh=mesh,
        out_type=jax.ShapeDtypeStruct((B, D), jnp.float32),
        scratch_types=[
            pltpu.VMEM((b_per_w,), jnp.int32),
            pltpu.VMEM((2, _CHUNK, 8, D), jnp.float32),
            pltpu.VMEM((2 * _CHUNK, D), jnp.float32),
            pltpu.SemaphoreType.DMA,
            pltpu.SemaphoreType.DMA,
            pltpu.SemaphoreType.DMA,
            pltpu.SemaphoreType.DMA,
        ],
    )
    def gather_kernel(idx_hbm, table_hbm, out_hbm, idx_v, land_v, row_v,
                      isem, gsem0, gsem1, osem):
        gsems = (gsem0, gsem1)
        wid = lax.axis_index("s") * _NUM_CORES + lax.axis_index("c")
        base = wid * b_per_w
        pltpu.async_copy(idx_hbm.at[pl.ds(base, b_per_w)], idx_v, isem).wait()

        def fire(c, buf):
            for g in range(_CHUNK // _L):
                grpv = jax.lax.shift_right_logical(
                    idx_v[pl.ds(c * _CHUNK + g * _L, _L)], 3
                )
                for j in range(_L):
                    pltpu.async_copy(
                        table_hbm.at[grpv[j]],
                        land_v.at[buf, g * _L + j],
                        gsems[buf],
                    )

        def drain(buf):
            for _ in range(_CHUNK):
                pltpu.make_async_copy(
                    table_hbm.at[0], land_v.at[0, 0], gsems[buf]
                ).wait()

        def owait():
            pltpu.make_async_copy(
                row_v, out_hbm.at[pl.ds(0, 2 * _CHUNK)], osem
            ).wait()

        def select(c, buf, half):
            for g in range(_CHUNK // _L):
                tokv = idx_v[pl.ds(c * _CHUNK + g * _L, _L)]
                rv = tokv & 7
                for j in range(_L):
                    r = rv[j]
                    t = half * _CHUNK + g * _L + j
                    for k in range(D // _L):
                        row_v[t, pl.ds(k * _L, _L)] = land_v[
                            buf, g * _L + j, r, pl.ds(k * _L, _L)
                        ]

        fire(0, 0)
        fire(1, 1)

        def body(i, _):
            @pl.when(i >= 1)
            def _():
                owait()

            drain(0)
            select(2 * i, 0, 0)

            @pl.when(i < n_iters - 1)
            def _():
                fire(2 * i + 2, 0)

            drain(1)
            select(2 * i + 1, 1, 1)

            @pl.when(i < n_iters - 1)
            def _():
                fire(2 * i + 3, 1)

            dst = pl.multiple_of(base + i * 2 * _CHUNK, 8)
            pltpu.async_copy(
                row_v, out_hbm.at[pl.ds(dst, 2 * _CHUNK)], osem
            )
            return 0

        lax.fori_loop(0, n_iters, body, 0)
        owait()

    return gather_kernel


def kernel(token_ids, E):
    B = token_ids.shape[0]
    V, D = E.shape
    idx = token_ids.astype(jnp.int32)
    E3 = E.reshape(V // 8, 8, D)
    return _build(B, V, D)(idx, E3)


# speedup vs baseline: 2.3202x; 1.0090x over previous
import functools

import jax
import jax.numpy as jnp
from jax import lax
from jax.experimental import pallas as pl
from jax.experimental.pallas import tpu as pltpu
from jax.experimental.pallas import tpu_sc as plsc

_NUM_CORES = 2
_NUM_SUBCORES = 16
_NUM_WORKERS = _NUM_CORES * _NUM_SUBCORES
_L = 16
_CHUNK = 32


@functools.lru_cache(maxsize=None)
def _build(B, V, D):
    b_per_w = B // _NUM_WORKERS
    n_iters = b_per_w // (2 * _CHUNK)
    mesh = plsc.VectorSubcoreMesh(core_axis_name="c", subcore_axis_name="s")

    @functools.partial(
        pl.kernel,
        mesh=mesh,
        out_type=jax.ShapeDtypeStruct((B, D), jnp.float32),
        scratch_types=[
            pltpu.VMEM((b_per_w,), jnp.int32),
            pltpu.VMEM((2, _CHUNK, 8, D), jnp.float32),
            pltpu.VMEM((2 * _CHUNK, D), jnp.float32),
            pltpu.SemaphoreType.DMA,
            pltpu.SemaphoreType.DMA,
            pltpu.SemaphoreType.DMA,
            pltpu.SemaphoreType.DMA,
        ],
    )
    def gather_kernel(idx_hbm, table_hbm, out_hbm, idx_v, land_v, row_v,
                      isem, gsem0, gsem1, osem):
        gsems = (gsem0, gsem1)
        wid = lax.axis_index("s") * _NUM_CORES + lax.axis_index("c")
        base = wid * b_per_w
        pltpu.async_copy(idx_hbm.at[pl.ds(base, b_per_w)], idx_v, isem).wait()

        def fire(c, buf):
            for g in range(_CHUNK // _L):
                grpv = jax.lax.shift_right_logical(
                    idx_v[pl.ds(c * _CHUNK + g * _L, _L)], 3
                )
                for j in range(_L):
                    pltpu.async_copy(
                        table_hbm.at[grpv[j]],
                        land_v.at[buf, g * _L + j],
                        gsems[buf],
                    )

        def drain(buf):
            for _ in range(_CHUNK):
                pltpu.make_async_copy(
                    table_hbm.at[0], land_v.at[0, 0], gsems[buf]
                ).wait()

        def owait():
            pltpu.make_async_copy(
                row_v, out_hbm.at[pl.ds(0, 2 * _CHUNK)], osem
            ).wait()

        def select(c, buf, half):
            for g in range(_CHUNK // _L):
                tokv = idx_v[pl.ds(c * _CHUNK + g * _L, _L)]
                rv = tokv & 7
                for j in range(_L):
                    r = rv[j]
                    t = half * _CHUNK + g * _L + j
                    for k in range(D // _L):
                        row_v[t, pl.ds(k * _L, _L)] = land_v[
                            buf, g * _L + j, r, pl.ds(k * _L, _L)
                        ]

        fire(0, 0)
        fire(1, 1)

        def body(i, _):
            @pl.when(i >= 1)
            def _():
                owait()

            drain(0)
            select(2 * i, 0, 0)

            @pl.when(i < n_iters - 1)
            def _():
                fire(2 * i + 2, 0)

            drain(1)
            select(2 * i + 1, 1, 1)

            @pl.when(i < n_iters - 1)
            def _():
                fire(2 * i + 3, 1)

            dst = pl.multiple_of(base + i * 2 * _CHUNK, 8)
            pltpu.async_copy(
                row_v, out_hbm.at[pl.ds(dst, 2 * _CHUNK)], osem
            )
            return 0

        lax.fori_loop(0, n_iters, body, 0)
        owait()

    return gather_kernel


def kernel(token_ids, E):
    B = token_ids.shape[0]
    V, D = E.shape
    idx = token_ids.astype(jnp.int32)
    E3 = E.reshape(V // 8, 8, D)
    return _build(B, V, D)(idx, E3)
